# baseline (speedup 1.0000x reference)
import jax
import jax.numpy as jnp
from jax import lax
from jax.experimental import pallas as pl
from jax.experimental.pallas import tpu as pltpu
from jax.experimental.pallas import tpu_sc as plsc

_NUM_WORKERS = 32
_CHUNK = 16
_NBUF = 3


def _gather_rows(table, flat_ids):
    n_ids = flat_ids.shape[0]
    d = table.shape[1]
    b_per_w = n_ids // _NUM_WORKERS
    n_chunks = b_per_w // _CHUNK
    assert b_per_w * _NUM_WORKERS == n_ids and n_chunks * _CHUNK == b_per_w
    assert n_chunks >= 2 * _NBUF
    mesh = plsc.VectorSubcoreMesh(core_axis_name="core",
                                  subcore_axis_name="subcore")

    row_buf = pltpu.VMEM((_CHUNK, d), table.dtype)

    @pl.kernel(
        out_type=jax.ShapeDtypeStruct((n_ids, d), table.dtype),
        mesh=mesh,
        scratch_types=[
            pltpu.VMEM((b_per_w,), jnp.int32),
            row_buf, row_buf, row_buf,
            pltpu.SemaphoreType.DMA, pltpu.SemaphoreType.DMA,
            pltpu.SemaphoreType.DMA, pltpu.SemaphoreType.DMA,
            pltpu.SemaphoreType.DMA, pltpu.SemaphoreType.DMA,
        ],
    )
    def gather_kernel(table_hbm, ids_hbm, out_hbm, idx_v,
                      buf0, buf1, buf2, g0, g1, g2, o0, o1, o2):
        bufs = (buf0, buf1, buf2)
        gsems = (g0, g1, g2)
        osems = (o0, o1, o2)
        wid = lax.axis_index("subcore") * 2 + lax.axis_index("core")
        base = wid * b_per_w
        pltpu.sync_copy(ids_hbm.at[pl.ds(base, b_per_w)], idx_v)

        def start_g(c, b):
            pltpu.async_copy(table_hbm.at[idx_v.at[pl.ds(c * _CHUNK, _CHUNK)]],
                             bufs[b], gsems[b])

        def wait_g(b):
            pltpu.make_async_copy(table_hbm.at[pl.ds(0, _CHUNK)], bufs[b],
                                  gsems[b]).wait()

        def start_w(c, b):
            pltpu.async_copy(bufs[b],
                             out_hbm.at[pl.ds(base + c * _CHUNK, _CHUNK)],
                             osems[b])

        def wait_w(b):
            pltpu.make_async_copy(
                bufs[b], out_hbm.at[pl.ds(base, _CHUNK)], osems[b]).wait()

        start_g(0, 0)
        start_g(1, 1)

        n_main = (n_chunks // _NBUF) * _NBUF

        @pl.loop(0, n_main, step=_NBUF)
        def _(k):
            for j in range(_NBUF):
                c = k + j
                b = j
                bp = (j + 2) % _NBUF
                wait_g(b)
                start_w(c, b)
                prefetch = c + 2 < n_chunks

                @pl.when(jnp.logical_and(prefetch, c >= 1))
                def _():
                    wait_w(bp)
                    start_g(c + 2, bp)

                @pl.when(jnp.logical_and(prefetch, c < 1))
                def _():
                    start_g(c + 2, bp)

        for c in range(n_main, n_chunks):
            b = c % _NBUF
            wait_g(b)
            start_w(c, b)

        for j in range(_NBUF):
            wait_w(j)

    return gather_kernel(table, flat_ids)


def kernel(input_ids, image_features, embed_tokens):
    del image_features
    batch, seq = input_ids.shape
    flat_ids = input_ids.reshape(batch * seq).astype(jnp.int32)
    rows = _gather_rows(embed_tokens, flat_ids)
    return rows.reshape(batch, seq, embed_tokens.shape[1])
